# baseline (speedup 1.0000x reference)
import functools

import jax
import jax.numpy as jnp
from jax import lax
from jax.experimental import pallas as pl
from jax.experimental.pallas import tpu as pltpu
from jax.experimental.pallas import tpu_sc as plsc

NC = 2
NS = 16
NW = NC * NS
CH = 128
BN_EPS = 1e-3


def _make_gather(B, K):
    b_per_w = B // NW
    n_chunks = b_per_w // CH
    n_tasks = 4 * n_chunks
    mesh = plsc.VectorSubcoreMesh(core_axis_name="c", subcore_axis_name="s")

    @functools.partial(
        pl.kernel,
        mesh=mesh,
        out_type=jax.ShapeDtypeStruct((B, 4 * K), jnp.float32),
        scratch_types=[
            pltpu.VMEM((b_per_w,), jnp.int32),
            pltpu.VMEM((b_per_w,), jnp.int32),
        ] + [pltpu.VMEM((CH, K), jnp.float32)] * 4
          + [pltpu.SemaphoreType.DMA] * 8,
    )
    def gather_kernel(u_hbm, m_hbm, Eu, Em, Eub, Emb, out,
                      idxu, idxm, r0, r1, r2, r3,
                      g0, g1, g2, g3, s0, s1, s2, s3):
        wid = lax.axis_index("s") * NC + lax.axis_index("c")
        base = wid * b_per_w
        cu = pltpu.async_copy(u_hbm.at[pl.ds(base, b_per_w)], idxu, g0)
        cm = pltpu.async_copy(m_hbm.at[pl.ds(base, b_per_w)], idxm, g1)
        cu.wait()
        cm.wait()
        tables = (Eu, Em, Eub, Emb)
        idxs = (idxu, idxm, idxu, idxm)
        rows = (r0, r1, r2, r3)
        gsem = (g0, g1, g2, g3)
        ssem = (s0, s1, s2, s3)
        NBUF = 4

        def task(t):
            ti, ci = divmod(t, n_chunks)
            return tables[ti], idxs[ti], ti, ci

        def fire_gather(t):
            table, idx, ti, ci = task(t)
            b = t % NBUF
            return pltpu.async_copy(
                table.at[idx.at[pl.ds(ci * CH, CH)]], rows[b], gsem[b])

        def fire_scatter(t):
            _, _, ti, ci = task(t)
            b = t % NBUF
            return pltpu.async_copy(
                rows[b],
                out.at[pl.ds(base + ci * CH, CH), pl.ds(ti * K, K)],
                ssem[b])

        depth = min(NBUF, n_tasks)
        gathers = [None] * n_tasks
        scatters = [None] * n_tasks
        for t in range(depth):
            gathers[t] = fire_gather(t)
        for t in range(n_tasks):
            gathers[t].wait()
            scatters[t] = fire_scatter(t)
            nxt = t + depth
            if nxt < n_tasks:
                scatters[t].wait()
                gathers[nxt] = fire_gather(nxt)
        for t in range(max(0, n_tasks - depth), n_tasks):
            scatters[t].wait()

    return gather_kernel


def _dense_body(g_ref, wh_ref, wl_ref, nb1_ref, sp_ref, sn_ref, cp_ref,
                cn_ref, fh_ref, fl_ref, c_ref, prev_ref, o_ref):
    del prev_ref
    H = 32
    x = g_ref[...]
    xh = x.astype(jnp.bfloat16)
    xl = (x - xh.astype(jnp.float32)).astype(jnp.bfloat16)
    y = jnp.dot(xh, wh_ref[...], preferred_element_type=jnp.float32)
    y += jnp.dot(xh, wl_ref[...], preferred_element_type=jnp.float32)
    y += jnp.dot(xl, wh_ref[...], preferred_element_type=jnp.float32)
    z = jnp.where(y >= nb1_ref[...], y * sp_ref[...] + cp_ref[...],
                  y * sn_ref[...] + cn_ref[...])
    zr = pltpu.roll(z, z.shape[1] - H, axis=1)
    lane = lax.broadcasted_iota(jnp.int32, z.shape, 1)
    combined = jnp.where(lane < H, z * zr, z)
    ch = combined.astype(jnp.bfloat16)
    cl = (combined - ch.astype(jnp.float32)).astype(jnp.bfloat16)
    acc = jnp.dot(ch, fh_ref[...], preferred_element_type=jnp.float32)
    acc += jnp.dot(ch, fl_ref[...], preferred_element_type=jnp.float32)
    acc += jnp.dot(cl, fh_ref[...], preferred_element_type=jnp.float32)
    o_ref[...] = acc + c_ref[0, 0]


def kernel(input_user, input_movie, Eu, Em, Eub, Emb, W1, b1, alpha, gamma,
           beta, mmean, mvar, W2, b2):
    B = input_user.shape[0]
    K = Eu.shape[1]
    H = W1.shape[2]

    u = input_user[:, 0].astype(jnp.int32)
    m = input_movie[:, 0].astype(jnp.int32)

    s = gamma * lax.rsqrt(mvar + BN_EPS)
    sn = alpha * s
    t = beta - mmean * s
    F = 4 * H
    spc = s.reshape(1, F)
    snc = sn.reshape(1, F)
    cpc = (s * b1 + t).reshape(1, F)
    cnc = (sn * b1 + t).reshape(1, F)
    nb1 = (-b1).reshape(1, F)
    Wblk = jnp.zeros((4, K, 4, H), jnp.float32)
    for i in range(4):
        Wblk = Wblk.at[i, :, i, :].set(W1[i])
    Wblk = Wblk.reshape(4 * K, F)
    Wh = Wblk.astype(jnp.bfloat16)
    Wl = (Wblk - Wh.astype(jnp.float32)).astype(jnp.bfloat16)
    wfin = jnp.concatenate(
        [jnp.ones((H, 1), jnp.float32), jnp.zeros((H, 1), jnp.float32),
         W2[0], W2[1]], axis=0)
    fh = wfin.astype(jnp.bfloat16)
    fl = (wfin - fh.astype(jnp.float32)).astype(jnp.bfloat16)
    cbias = (b2[0, 0] + b2[1, 0]).reshape(1, 1)

    NCHUNK = 2
    Bc = B // NCHUNK
    R = 2048
    gather_fn = _make_gather(Bc, K)

    nsteps = Bc // R

    def dense_fn(g, c, prev):
        return pl.pallas_call(
            _dense_body,
            grid=(nsteps,),
            in_specs=[
                pl.BlockSpec((R, 4 * K), lambda i: (i, 0)),
                pl.BlockSpec((4 * K, F), lambda i: (0, 0)),
                pl.BlockSpec((4 * K, F), lambda i: (0, 0)),
                pl.BlockSpec((1, F), lambda i: (0, 0)),
                pl.BlockSpec((1, F), lambda i: (0, 0)),
                pl.BlockSpec((1, F), lambda i: (0, 0)),
                pl.BlockSpec((1, F), lambda i: (0, 0)),
                pl.BlockSpec((1, F), lambda i: (0, 0)),
                pl.BlockSpec((F, 1), lambda i: (0, 0)),
                pl.BlockSpec((F, 1), lambda i: (0, 0)),
                pl.BlockSpec((1, 1), lambda i: (0, 0),
                             memory_space=pltpu.SMEM),
                pl.BlockSpec(memory_space=pl.ANY),
            ],
            out_specs=pl.BlockSpec((R, 1),
                                   lambda i, c=c: (c * nsteps + i, 0)),
            out_shape=jax.ShapeDtypeStruct((B, 1), jnp.float32),
            input_output_aliases={11: 0},
        )(g, Wh, Wl, nb1, spc, snc, cpc, cnc, fh, fl, cbias, prev)

    out = jnp.zeros((B, 1), jnp.float32)
    gs = []
    for c in range(NCHUNK):
        sl = slice(c * Bc, (c + 1) * Bc)
        gs.append(gather_fn(u[sl], m[sl], Eu, Em, Eub, Emb))
    for c in range(NCHUNK):
        out = dense_fn(gs[c], c, out)
    return out


# speedup vs baseline: 1.0367x; 1.0367x over previous
import functools

import jax
import jax.numpy as jnp
from jax import lax
from jax.experimental import pallas as pl
from jax.experimental.pallas import tpu as pltpu
from jax.experimental.pallas import tpu_sc as plsc

NC = 2
NS = 16
NW = NC * NS
CH = 128
BN_EPS = 1e-3


def _make_gather(B, K):
    b_per_w = B // NW
    n_chunks = b_per_w // CH
    n_tasks = 4 * n_chunks
    mesh = plsc.VectorSubcoreMesh(core_axis_name="c", subcore_axis_name="s")

    @functools.partial(
        pl.kernel,
        mesh=mesh,
        out_type=jax.ShapeDtypeStruct((B, 4 * K), jnp.float32),
        scratch_types=[
            pltpu.VMEM((b_per_w,), jnp.int32),
            pltpu.VMEM((b_per_w,), jnp.int32),
        ] + [pltpu.VMEM((CH, K), jnp.float32)] * 4
          + [pltpu.SemaphoreType.DMA] * 8,
    )
    def gather_kernel(u_hbm, m_hbm, Eu, Em, Eub, Emb, out,
                      idxu, idxm, r0, r1, r2, r3,
                      g0, g1, g2, g3, s0, s1, s2, s3):
        wid = lax.axis_index("s") * NC + lax.axis_index("c")
        base = wid * b_per_w
        cu = pltpu.async_copy(u_hbm.at[pl.ds(base, b_per_w)], idxu, g0)
        cm = pltpu.async_copy(m_hbm.at[pl.ds(base, b_per_w)], idxm, g1)
        cu.wait()
        cm.wait()
        tables = (Eu, Em, Eub, Emb)
        idxs = (idxu, idxm, idxu, idxm)
        rows = (r0, r1, r2, r3)
        gsem = (g0, g1, g2, g3)
        ssem = (s0, s1, s2, s3)
        NBUF = 4

        def task(t):
            ti, ci = divmod(t, n_chunks)
            return tables[ti], idxs[ti], ti, ci

        def fire_gather(t):
            table, idx, ti, ci = task(t)
            b = t % NBUF
            return pltpu.async_copy(
                table.at[idx.at[pl.ds(ci * CH, CH)]], rows[b], gsem[b])

        def fire_scatter(t):
            _, _, ti, ci = task(t)
            b = t % NBUF
            return pltpu.async_copy(
                rows[b],
                out.at[pl.ds(base + ci * CH, CH), pl.ds(ti * K, K)],
                ssem[b])

        depth = min(NBUF, n_tasks)
        gathers = [None] * n_tasks
        scatters = [None] * n_tasks
        for t in range(depth):
            gathers[t] = fire_gather(t)
        for t in range(n_tasks):
            gathers[t].wait()
            scatters[t] = fire_scatter(t)
            nxt = t + depth
            if nxt < n_tasks:
                scatters[t].wait()
                gathers[nxt] = fire_gather(nxt)
        for t in range(max(0, n_tasks - depth), n_tasks):
            scatters[t].wait()

    return gather_kernel


def _dense_body(g_ref, wh_ref, wl_ref, nb1_ref, sp_ref, sn_ref, cp_ref,
                cn_ref, fh_ref, fl_ref, c_ref, o_ref):
    H = 32
    x = g_ref[...]
    xh = x.astype(jnp.bfloat16)
    xl = (x - xh.astype(jnp.float32)).astype(jnp.bfloat16)
    y = jnp.dot(xh, wh_ref[...], preferred_element_type=jnp.float32)
    y += jnp.dot(xh, wl_ref[...], preferred_element_type=jnp.float32)
    y += jnp.dot(xl, wh_ref[...], preferred_element_type=jnp.float32)
    z = jnp.where(y >= nb1_ref[...], y * sp_ref[...] + cp_ref[...],
                  y * sn_ref[...] + cn_ref[...])
    zr = pltpu.roll(z, z.shape[1] - H, axis=1)
    lane = lax.broadcasted_iota(jnp.int32, z.shape, 1)
    combined = jnp.where(lane < H, z * zr, z)
    ch = combined.astype(jnp.bfloat16)
    cl = (combined - ch.astype(jnp.float32)).astype(jnp.bfloat16)
    acc = jnp.dot(ch, fh_ref[...], preferred_element_type=jnp.float32)
    acc += jnp.dot(ch, fl_ref[...], preferred_element_type=jnp.float32)
    acc += jnp.dot(cl, fh_ref[...], preferred_element_type=jnp.float32)
    o_ref[...] = acc + c_ref[0, 0]


def kernel(input_user, input_movie, Eu, Em, Eub, Emb, W1, b1, alpha, gamma,
           beta, mmean, mvar, W2, b2):
    B = input_user.shape[0]
    K = Eu.shape[1]
    H = W1.shape[2]

    u = input_user[:, 0].astype(jnp.int32)
    m = input_movie[:, 0].astype(jnp.int32)

    s = gamma * lax.rsqrt(mvar + BN_EPS)
    sn = alpha * s
    t = beta - mmean * s
    F = 4 * H
    spc = s.reshape(1, F)
    snc = sn.reshape(1, F)
    cpc = (s * b1 + t).reshape(1, F)
    cnc = (sn * b1 + t).reshape(1, F)
    nb1 = (-b1).reshape(1, F)
    Wblk = jnp.zeros((4, K, 4, H), jnp.float32)
    for i in range(4):
        Wblk = Wblk.at[i, :, i, :].set(W1[i])
    Wblk = Wblk.reshape(4 * K, F)
    Wh = Wblk.astype(jnp.bfloat16)
    Wl = (Wblk - Wh.astype(jnp.float32)).astype(jnp.bfloat16)
    wfin = jnp.concatenate(
        [jnp.ones((H, 1), jnp.float32), jnp.zeros((H, 1), jnp.float32),
         W2[0], W2[1]], axis=0)
    fh = wfin.astype(jnp.bfloat16)
    fl = (wfin - fh.astype(jnp.float32)).astype(jnp.bfloat16)
    cbias = (b2[0, 0] + b2[1, 0]).reshape(1, 1)

    NCHUNK = 2
    Bc = B // NCHUNK
    R = 2048
    gather_fn = _make_gather(Bc, K)

    def dense_fn(g):
        return pl.pallas_call(
            _dense_body,
            grid=(Bc // R,),
            in_specs=[
                pl.BlockSpec((R, 4 * K), lambda i: (i, 0)),
                pl.BlockSpec((4 * K, F), lambda i: (0, 0)),
                pl.BlockSpec((4 * K, F), lambda i: (0, 0)),
                pl.BlockSpec((1, F), lambda i: (0, 0)),
                pl.BlockSpec((1, F), lambda i: (0, 0)),
                pl.BlockSpec((1, F), lambda i: (0, 0)),
                pl.BlockSpec((1, F), lambda i: (0, 0)),
                pl.BlockSpec((1, F), lambda i: (0, 0)),
                pl.BlockSpec((F, 1), lambda i: (0, 0)),
                pl.BlockSpec((F, 1), lambda i: (0, 0)),
                pl.BlockSpec((1, 1), lambda i: (0, 0),
                             memory_space=pltpu.SMEM),
            ],
            out_specs=pl.BlockSpec((R, 1), lambda i: (i, 0)),
            out_shape=jax.ShapeDtypeStruct((Bc, 1), jnp.float32),
        )(g, Wh, Wl, nb1, spc, snc, cpc, cnc, fh, fl, cbias)

    outs = []
    for c in range(NCHUNK):
        sl = slice(c * Bc, (c + 1) * Bc)
        g = gather_fn(u[sl], m[sl], Eu, Em, Eub, Emb)
        outs.append(dense_fn(g))
    return jnp.concatenate(outs, axis=0)


# speedup vs baseline: 1.0410x; 1.0042x over previous
import functools

import jax
import jax.numpy as jnp
from jax import lax
from jax.experimental import pallas as pl
from jax.experimental.pallas import tpu as pltpu
from jax.experimental.pallas import tpu_sc as plsc

NC = 2
NS = 16
NW = NC * NS
CH = 128
BN_EPS = 1e-3


def _make_gather(B, K):
    b_per_w = B // NW
    n_chunks = b_per_w // CH
    n_tasks = 4 * n_chunks
    mesh = plsc.VectorSubcoreMesh(core_axis_name="c", subcore_axis_name="s")

    @functools.partial(
        pl.kernel,
        mesh=mesh,
        out_type=jax.ShapeDtypeStruct((B, 4 * K), jnp.float32),
        scratch_types=[
            pltpu.VMEM((b_per_w,), jnp.int32),
            pltpu.VMEM((b_per_w,), jnp.int32),
        ] + [pltpu.VMEM((CH, K), jnp.float32)] * 6
          + [pltpu.SemaphoreType.DMA] * 12,
    )
    def gather_kernel(u_hbm, m_hbm, Eu, Em, Eub, Emb, out,
                      idxu, idxm, r0, r1, r2, r3, r4, r5,
                      g0, g1, g2, g3, g4, g5, s0, s1, s2, s3, s4, s5):
        wid = lax.axis_index("s") * NC + lax.axis_index("c")
        base = wid * b_per_w
        cu = pltpu.async_copy(u_hbm.at[pl.ds(base, b_per_w)], idxu, g0)
        cm = pltpu.async_copy(m_hbm.at[pl.ds(base, b_per_w)], idxm, g1)
        cu.wait()
        cm.wait()
        tables = (Eu, Em, Eub, Emb)
        idxs = (idxu, idxm, idxu, idxm)
        rows = (r0, r1, r2, r3, r4, r5)
        gsem = (g0, g1, g2, g3, g4, g5)
        ssem = (s0, s1, s2, s3, s4, s5)
        NBUF = 6

        def task(t):
            ti, ci = divmod(t, n_chunks)
            return tables[ti], idxs[ti], ti, ci

        def fire_gather(t):
            table, idx, ti, ci = task(t)
            b = t % NBUF
            return pltpu.async_copy(
                table.at[idx.at[pl.ds(ci * CH, CH)]], rows[b], gsem[b])

        def fire_scatter(t):
            _, _, ti, ci = task(t)
            b = t % NBUF
            return pltpu.async_copy(
                rows[b],
                out.at[pl.ds(base + ci * CH, CH), pl.ds(ti * K, K)],
                ssem[b])

        depth = min(NBUF, n_tasks)
        gathers = [None] * n_tasks
        scatters = [None] * n_tasks
        for t in range(depth):
            gathers[t] = fire_gather(t)
        for t in range(n_tasks):
            gathers[t].wait()
            scatters[t] = fire_scatter(t)
            nxt = t + depth
            if nxt < n_tasks:
                scatters[t].wait()
                gathers[nxt] = fire_gather(nxt)
        for t in range(max(0, n_tasks - depth), n_tasks):
            scatters[t].wait()

    return gather_kernel


def _dense_body(g_ref, wh_ref, wl_ref, nb1_ref, sp_ref, sn_ref, cp_ref,
                cn_ref, fh_ref, fl_ref, c_ref, o_ref):
    H = 32
    x = g_ref[...]
    xh = x.astype(jnp.bfloat16)
    xl = (x - xh.astype(jnp.float32)).astype(jnp.bfloat16)
    y = jnp.dot(xh, wh_ref[...], preferred_element_type=jnp.float32)
    y += jnp.dot(xh, wl_ref[...], preferred_element_type=jnp.float32)
    y += jnp.dot(xl, wh_ref[...], preferred_element_type=jnp.float32)
    z = jnp.where(y >= nb1_ref[...], y * sp_ref[...] + cp_ref[...],
                  y * sn_ref[...] + cn_ref[...])
    zr = pltpu.roll(z, z.shape[1] - H, axis=1)
    lane = lax.broadcasted_iota(jnp.int32, z.shape, 1)
    combined = jnp.where(lane < H, z * zr, z)
    ch = combined.astype(jnp.bfloat16)
    cl = (combined - ch.astype(jnp.float32)).astype(jnp.bfloat16)
    acc = jnp.dot(ch, fh_ref[...], preferred_element_type=jnp.float32)
    acc += jnp.dot(ch, fl_ref[...], preferred_element_type=jnp.float32)
    acc += jnp.dot(cl, fh_ref[...], preferred_element_type=jnp.float32)
    o_ref[...] = acc + c_ref[0, 0]


def kernel(input_user, input_movie, Eu, Em, Eub, Emb, W1, b1, alpha, gamma,
           beta, mmean, mvar, W2, b2):
    B = input_user.shape[0]
    K = Eu.shape[1]
    H = W1.shape[2]

    u = input_user[:, 0].astype(jnp.int32)
    m = input_movie[:, 0].astype(jnp.int32)

    s = gamma * lax.rsqrt(mvar + BN_EPS)
    sn = alpha * s
    t = beta - mmean * s
    F = 4 * H
    spc = s.reshape(1, F)
    snc = sn.reshape(1, F)
    cpc = (s * b1 + t).reshape(1, F)
    cnc = (sn * b1 + t).reshape(1, F)
    nb1 = (-b1).reshape(1, F)
    Wblk = jnp.zeros((4, K, 4, H), jnp.float32)
    for i in range(4):
        Wblk = Wblk.at[i, :, i, :].set(W1[i])
    Wblk = Wblk.reshape(4 * K, F)
    Wh = Wblk.astype(jnp.bfloat16)
    Wl = (Wblk - Wh.astype(jnp.float32)).astype(jnp.bfloat16)
    wfin = jnp.concatenate(
        [jnp.ones((H, 1), jnp.float32), jnp.zeros((H, 1), jnp.float32),
         W2[0], W2[1]], axis=0)
    fh = wfin.astype(jnp.bfloat16)
    fl = (wfin - fh.astype(jnp.float32)).astype(jnp.bfloat16)
    cbias = (b2[0, 0] + b2[1, 0]).reshape(1, 1)

    NCHUNK = 2
    Bc = B // NCHUNK
    R = 2048
    gather_fn = _make_gather(Bc, K)

    def dense_fn(g):
        return pl.pallas_call(
            _dense_body,
            grid=(Bc // R,),
            in_specs=[
                pl.BlockSpec((R, 4 * K), lambda i: (i, 0)),
                pl.BlockSpec((4 * K, F), lambda i: (0, 0)),
                pl.BlockSpec((4 * K, F), lambda i: (0, 0)),
                pl.BlockSpec((1, F), lambda i: (0, 0)),
                pl.BlockSpec((1, F), lambda i: (0, 0)),
                pl.BlockSpec((1, F), lambda i: (0, 0)),
                pl.BlockSpec((1, F), lambda i: (0, 0)),
                pl.BlockSpec((1, F), lambda i: (0, 0)),
                pl.BlockSpec((F, 1), lambda i: (0, 0)),
                pl.BlockSpec((F, 1), lambda i: (0, 0)),
                pl.BlockSpec((1, 1), lambda i: (0, 0),
                             memory_space=pltpu.SMEM),
            ],
            out_specs=pl.BlockSpec((R, 1), lambda i: (i, 0)),
            out_shape=jax.ShapeDtypeStruct((Bc, 1), jnp.float32),
        )(g, Wh, Wl, nb1, spc, snc, cpc, cnc, fh, fl, cbias)

    outs = []
    for c in range(NCHUNK):
        sl = slice(c * Bc, (c + 1) * Bc)
        g = gather_fn(u[sl], m[sl], Eu, Em, Eub, Emb)
        outs.append(dense_fn(g))
    return jnp.concatenate(outs, axis=0)


# speedup vs baseline: 1.0522x; 1.0107x over previous
import functools

import jax
import jax.numpy as jnp
from jax import lax
from jax.experimental import pallas as pl
from jax.experimental.pallas import tpu as pltpu
from jax.experimental.pallas import tpu_sc as plsc

NC = 2
NS = 16
NW = NC * NS
CH = 128
BN_EPS = 1e-3


def _make_gather(B, K):
    b_per_w = B // NW
    n_chunks = b_per_w // CH
    n_tasks = 4 * n_chunks
    mesh = plsc.VectorSubcoreMesh(core_axis_name="c", subcore_axis_name="s")

    @functools.partial(
        pl.kernel,
        mesh=mesh,
        out_type=jax.ShapeDtypeStruct((B, 4 * K), jnp.float32),
        scratch_types=[
            pltpu.VMEM((b_per_w,), jnp.int32),
            pltpu.VMEM((b_per_w,), jnp.int32),
        ] + [pltpu.VMEM((CH, K), jnp.float32)] * 6
          + [pltpu.SemaphoreType.DMA] * 12,
    )
    def gather_kernel(u_hbm, m_hbm, Eu, Em, Eub, Emb, out,
                      idxu, idxm, r0, r1, r2, r3, r4, r5,
                      g0, g1, g2, g3, g4, g5, s0, s1, s2, s3, s4, s5):
        wid = lax.axis_index("s") * NC + lax.axis_index("c")
        base = wid * b_per_w
        cu = pltpu.async_copy(u_hbm.at[pl.ds(base, b_per_w)], idxu, g0)
        cm = pltpu.async_copy(m_hbm.at[pl.ds(base, b_per_w)], idxm, g1)
        cu.wait()
        cm.wait()
        tables = (Eu, Em, Eub, Emb)
        idxs = (idxu, idxm, idxu, idxm)
        rows = (r0, r1, r2, r3, r4, r5)
        gsem = (g0, g1, g2, g3, g4, g5)
        ssem = (s0, s1, s2, s3, s4, s5)
        NBUF = 6

        def task(t):
            ti, ci = divmod(t, n_chunks)
            return tables[ti], idxs[ti], ti, ci

        def fire_gather(t):
            table, idx, ti, ci = task(t)
            b = t % NBUF
            return pltpu.async_copy(
                table.at[idx.at[pl.ds(ci * CH, CH)]], rows[b], gsem[b])

        def fire_scatter(t):
            _, _, ti, ci = task(t)
            b = t % NBUF
            return pltpu.async_copy(
                rows[b],
                out.at[pl.ds(base + ci * CH, CH), pl.ds(ti * K, K)],
                ssem[b])

        depth = min(NBUF, n_tasks)
        gathers = [None] * n_tasks
        scatters = [None] * n_tasks
        for t in range(depth):
            gathers[t] = fire_gather(t)
        for t in range(n_tasks):
            gathers[t].wait()
            scatters[t] = fire_scatter(t)
            nxt = t + depth
            if nxt < n_tasks:
                scatters[t].wait()
                gathers[nxt] = fire_gather(nxt)
        for t in range(max(0, n_tasks - depth), n_tasks):
            scatters[t].wait()

    return gather_kernel


def _dense_body(g_ref, wh_ref, wl_ref, nb1_ref, sp_ref, sn_ref, cp_ref,
                cn_ref, fh_ref, fl_ref, c_ref, o_ref):
    H = 32
    x = g_ref[...]
    xh = x.astype(jnp.bfloat16)
    xl = (x - xh.astype(jnp.float32)).astype(jnp.bfloat16)
    y = jnp.dot(xh, wh_ref[...], preferred_element_type=jnp.float32)
    y += jnp.dot(xh, wl_ref[...], preferred_element_type=jnp.float32)
    y += jnp.dot(xl, wh_ref[...], preferred_element_type=jnp.float32)
    z = jnp.where(y >= nb1_ref[...], y * sp_ref[...] + cp_ref[...],
                  y * sn_ref[...] + cn_ref[...])
    zr = pltpu.roll(z, z.shape[1] - H, axis=1)
    lane = lax.broadcasted_iota(jnp.int32, z.shape, 1)
    combined = jnp.where(lane < H, z * zr, z)
    ch = combined.astype(jnp.bfloat16)
    cl = (combined - ch.astype(jnp.float32)).astype(jnp.bfloat16)
    acc = jnp.dot(ch, fh_ref[...], preferred_element_type=jnp.float32)
    acc += jnp.dot(ch, fl_ref[...], preferred_element_type=jnp.float32)
    acc += jnp.dot(cl, fh_ref[...], preferred_element_type=jnp.float32)
    o_ref[...] = acc + c_ref[0, 0]


def kernel(input_user, input_movie, Eu, Em, Eub, Emb, W1, b1, alpha, gamma,
           beta, mmean, mvar, W2, b2):
    B = input_user.shape[0]
    K = Eu.shape[1]
    H = W1.shape[2]

    u = input_user[:, 0].astype(jnp.int32)
    m = input_movie[:, 0].astype(jnp.int32)

    s = gamma * lax.rsqrt(mvar + BN_EPS)
    sn = alpha * s
    t = beta - mmean * s
    F = 4 * H
    spc = s.reshape(1, F)
    snc = sn.reshape(1, F)
    cpc = (s * b1 + t).reshape(1, F)
    cnc = (sn * b1 + t).reshape(1, F)
    nb1 = (-b1).reshape(1, F)
    Wblk = jnp.zeros((4, K, 4, H), jnp.float32)
    for i in range(4):
        Wblk = Wblk.at[i, :, i, :].set(W1[i])
    Wblk = Wblk.reshape(4 * K, F)
    Wh = Wblk.astype(jnp.bfloat16)
    Wl = (Wblk - Wh.astype(jnp.float32)).astype(jnp.bfloat16)
    wfin = jnp.concatenate(
        [jnp.ones((H, 1), jnp.float32), jnp.zeros((H, 1), jnp.float32),
         W2[0], W2[1]], axis=0)
    fh = wfin.astype(jnp.bfloat16)
    fl = (wfin - fh.astype(jnp.float32)).astype(jnp.bfloat16)
    cbias = (b2[0, 0] + b2[1, 0]).reshape(1, 1)

    NCHUNK = 2
    Bc = B // NCHUNK
    R = 4096
    gather_fn = _make_gather(Bc, K)

    def dense_fn(g):
        return pl.pallas_call(
            _dense_body,
            grid=(Bc // R,),
            in_specs=[
                pl.BlockSpec((R, 4 * K), lambda i: (i, 0)),
                pl.BlockSpec((4 * K, F), lambda i: (0, 0)),
                pl.BlockSpec((4 * K, F), lambda i: (0, 0)),
                pl.BlockSpec((1, F), lambda i: (0, 0)),
                pl.BlockSpec((1, F), lambda i: (0, 0)),
                pl.BlockSpec((1, F), lambda i: (0, 0)),
                pl.BlockSpec((1, F), lambda i: (0, 0)),
                pl.BlockSpec((1, F), lambda i: (0, 0)),
                pl.BlockSpec((F, 1), lambda i: (0, 0)),
                pl.BlockSpec((F, 1), lambda i: (0, 0)),
                pl.BlockSpec((1, 1), lambda i: (0, 0),
                             memory_space=pltpu.SMEM),
            ],
            out_specs=pl.BlockSpec((R, 1), lambda i: (i, 0)),
            out_shape=jax.ShapeDtypeStruct((Bc, 1), jnp.float32),
        )(g, Wh, Wl, nb1, spc, snc, cpc, cnc, fh, fl, cbias)

    outs = []
    for c in range(NCHUNK):
        sl = slice(c * Bc, (c + 1) * Bc)
        g = gather_fn(u[sl], m[sl], Eu, Em, Eub, Emb)
        outs.append(dense_fn(g))
    return jnp.concatenate(outs, axis=0)
